# Initial kernel scaffold; baseline (speedup 1.0000x reference)
import jax
import jax.numpy as jnp
from jax.experimental import pallas as pl


def kernel(input_ids, pos_table, glyph_table, graph_table, stroke_table, gamma, beta):
    raise NotImplementedError("write your pallas kernel here")


# speedup vs baseline: 1.0307x; 1.0307x over previous
import functools

import jax
import jax.numpy as jnp
from jax import lax
from jax.experimental import pallas as pl
from jax.experimental.pallas import tpu as pltpu
from jax.experimental.pallas import tpu_sc as plsc

D_GLYPH = 512
D = 1024
BATCH = 4
SEQ = 2048
TOK = BATCH * SEQ
NC = 2
NS = 16
NW = NC * NS
TPW = TOK // NW
CH = 32
NCH = TPW // CH
LN_EPS = 1e-12
VPG = D_GLYPH // 16


def _sc_gather_sum(ids, glyph, graph, stroke):
  mesh = plsc.VectorSubcoreMesh(core_axis_name="c", subcore_axis_name="s")

  @functools.partial(
      pl.kernel,
      mesh=mesh,
      out_type=[
          jax.ShapeDtypeStruct((TOK, D), jnp.float32),
          jax.ShapeDtypeStruct((NW, 16), jnp.float32),
      ],
      scratch_types=[
          pltpu.VMEM((CH,), jnp.int32),
          pltpu.VMEM((CH, D_GLYPH), jnp.float32),
          pltpu.VMEM((CH, D_GLYPH), jnp.float32),
          pltpu.VMEM((CH, D), jnp.float32),
          pltpu.VMEM((16,), jnp.float32),
          pltpu.SemaphoreType.DMA,
      ],
  )
  def body(ids_hbm, glyph_hbm, graph_hbm, stroke_hbm, sum_hbm, sq_hbm,
           idx_v, gly_v, gra_v, str_v, sq_v, sem):
    wid = lax.axis_index("s") * NC + lax.axis_index("c")
    base = wid * TPW

    def chunk(k, acc):
      off = base + k * CH
      pltpu.sync_copy(ids_hbm.at[pl.ds(off, CH)], idx_v)
      cg = pltpu.async_copy(glyph_hbm.at[idx_v], gly_v, sem)
      cr = pltpu.async_copy(graph_hbm.at[idx_v], gra_v, sem)
      cs = pltpu.async_copy(stroke_hbm.at[idx_v], str_v, sem)
      cg.wait()
      cr.wait()
      cs.wait()

      def inner(jj, a):
        t = jj // VPG
        j = (jj % VPG) * 16
        g = gly_v[t, pl.ds(j, 16)]
        str_v[t, pl.ds(j, 16)] = str_v[t, pl.ds(j, 16)] + g
        str_v[t, pl.ds(D_GLYPH + j, 16)] = (
            str_v[t, pl.ds(D_GLYPH + j, 16)] + gra_v[t, pl.ds(j, 16)])
        return a + g * g

      acc = lax.fori_loop(0, CH * VPG, inner, acc)
      pltpu.sync_copy(str_v, sum_hbm.at[pl.ds(off, CH)])
      return acc

    acc = lax.fori_loop(0, NCH, chunk, jnp.zeros((16,), jnp.float32))
    sq_v[...] = acc
    pltpu.sync_copy(sq_v, sq_hbm.at[wid])

  return body(ids, glyph, graph, stroke)


_RB = 256
_GRID = TOK // _RB


def _ln_body(sum_ref, pos_ref, gam_ref, bet_ref, sq_ref, out_ref, loss_ref):
  x = sum_ref[...] + pos_ref[...]
  m = jnp.mean(x, axis=-1, keepdims=True)
  v = jnp.mean((x - m) ** 2, axis=-1, keepdims=True)
  y = (x - m) / jnp.sqrt(v + LN_EPS)
  out_ref[...] = y * gam_ref[...] + bet_ref[...]

  @pl.when(pl.program_id(0) == 0)
  def _():
    loss_ref[...] = (jnp.sum(sq_ref[...]) / float(TOK * D_GLYPH)).reshape(1, 1)


def kernel(input_ids, pos_table, glyph_table, graph_table, stroke_table,
           gamma, beta):
  ids = input_ids.astype(jnp.int32).reshape(TOK)
  sum_flat, partials = _sc_gather_sum(
      ids, glyph_table, graph_table, stroke_table)

  emb, loss = pl.pallas_call(
      _ln_body,
      grid=(_GRID,),
      in_specs=[
          pl.BlockSpec((_RB, D), lambda i: (i, 0)),
          pl.BlockSpec((_RB, D), lambda i: (i % (SEQ // _RB), 0)),
          pl.BlockSpec((1, D), lambda i: (0, 0)),
          pl.BlockSpec((1, D), lambda i: (0, 0)),
          pl.BlockSpec((NW, 16), lambda i: (0, 0)),
      ],
      out_specs=[
          pl.BlockSpec((_RB, D), lambda i: (i, 0)),
          pl.BlockSpec((1, 1), lambda i: (0, 0)),
      ],
      out_shape=[
          jax.ShapeDtypeStruct((TOK, D), jnp.float32),
          jax.ShapeDtypeStruct((1, 1), jnp.float32),
      ],
  )(sum_flat, pos_table, gamma.reshape(1, D), beta.reshape(1, D), partials)

  return emb.reshape(BATCH, SEQ, D), loss[0, 0]
